# baseline (speedup 1.0000x reference)
import functools

import jax
import jax.numpy as jnp
from jax import lax
from jax.experimental import pallas as pl
from jax.experimental.pallas import tpu as pltpu
from jax.experimental.pallas import tpu_sc as plsc

D_MODEL = 1024
N_SCALES = 3
N_EXP = 4
N_TOTAL = 12
TOP_K = 2
D_FFN = 2752


def _logits_kernel(x_ref, wr_ref, lt_ref):
    logits = jnp.dot(x_ref[...], wr_ref[...],
                     preferred_element_type=jnp.float32)
    lt_ref[...] = logits.T


def _logits(x_flat, wr):
    n_tok = x_flat.shape[0]
    blk = 2048
    return pl.pallas_call(
        _logits_kernel,
        grid=(n_tok // blk,),
        in_specs=[
            pl.BlockSpec((blk, D_MODEL), lambda i: (i, 0)),
            pl.BlockSpec((D_MODEL, N_TOTAL), lambda i: (0, 0)),
        ],
        out_specs=pl.BlockSpec((N_TOTAL, blk), lambda i: (0, i)),
        out_shape=jax.ShapeDtypeStruct((N_TOTAL, n_tok), jnp.float32),
    )(x_flat, wr)


_SC_NW = 32
_SC_TPW = 128
_SC_G = _SC_TPW // 16


def _sc_router_body(lt_hbm, w_hbm, part_hbm, lbuf, wbuf, pbuf):
    c = lax.axis_index("c")
    s = lax.axis_index("s")
    wid = s * 2 + c
    t0 = pl.multiple_of(wid * _SC_TPW, _SC_TPW)
    pltpu.sync_copy(lt_hbm.at[:, pl.ds(t0, _SC_TPW)], lbuf)

    iota = lax.iota(jnp.int32, 16)
    acc_load = [jnp.zeros((16,), jnp.float32) for _ in range(N_TOTAL)]
    acc_cnt = [jnp.zeros((16,), jnp.float32) for _ in range(N_TOTAL)]
    for g in range(_SC_G):
        ls = [lbuf[e, pl.ds(g * 16, 16)] for e in range(N_TOTAL)]
        m = ls[0]
        for e in range(1, N_TOTAL):
            m = jnp.maximum(m, ls[e])
        es = [jnp.exp(ls[e] - m) for e in range(N_TOTAL)]
        z = es[0]
        for e in range(1, N_TOTAL):
            z = z + es[e]
        zi = 1.0 / z
        e1 = es[0]
        i1 = jnp.zeros((16,), jnp.int32)
        e2 = jnp.zeros((16,), jnp.float32)
        i2 = jnp.full((16,), N_TOTAL, jnp.int32)
        for e in range(1, N_TOTAL):
            v = es[e]
            gt1 = v > e1
            gt2 = v > e2
            i2 = jnp.where(gt1, i1, jnp.where(gt2, e, i2))
            e2 = jnp.where(gt1, e1, jnp.where(gt2, v, e2))
            i1 = jnp.where(gt1, e, i1)
            e1 = jnp.where(gt1, v, e1)
        d = e1 + e2 + 1e-8 * z
        w1 = e1 / d
        w2 = e2 / d
        for e in range(N_TOTAL):
            we = (jnp.where(i1 == e, w1, 0.0) + jnp.where(i2 == e, w2, 0.0))
            wbuf[e, pl.ds(g * 16, 16)] = we
            acc_load[e] = acc_load[e] + es[e] * zi
            acc_cnt[e] = (acc_cnt[e]
                          + jnp.where(i1 == e, 1.0, 0.0)
                          + jnp.where(i2 == e, 1.0, 0.0))

    for e in range(N_TOTAL):
        pbuf[0, pl.ds(e * 16, 16)] = acc_load[e]
        pbuf[1, pl.ds(e * 16, 16)] = acc_cnt[e]
    pltpu.sync_copy(wbuf, w_hbm.at[:, pl.ds(t0, _SC_TPW)])
    pltpu.sync_copy(pbuf, part_hbm.at[wid])


def _sc_router(ltr):
    n_tok = ltr.shape[1]
    return pl.kernel(
        _sc_router_body,
        out_type=[
            jax.ShapeDtypeStruct((N_TOTAL, n_tok), jnp.float32),
            jax.ShapeDtypeStruct((_SC_NW, 2, N_TOTAL * 16), jnp.float32),
        ],
        mesh=plsc.VectorSubcoreMesh(core_axis_name="c", subcore_axis_name="s",
                                    num_cores=2, num_subcores=16),
        scratch_types=[
            pltpu.VMEM((N_TOTAL, _SC_TPW), jnp.float32),
            pltpu.VMEM((N_TOTAL, _SC_TPW), jnp.float32),
            pltpu.VMEM((2, N_TOTAL * 16), jnp.float32),
        ],
    )(ltr)


def _pool_kernel(n_k, x_ref, wp_ref, o_ref, acc_ref):
    k = pl.program_id(0)

    @pl.when(k == 0)
    def _():
        acc_ref[...] = jnp.zeros_like(acc_ref)

    xb = x_ref[...].astype(jnp.bfloat16)
    wb = wp_ref[...].astype(jnp.bfloat16)
    acc_ref[...] += jnp.dot(xb, wb, preferred_element_type=jnp.float32)

    @pl.when(k == n_k - 1)
    def _():
        o_ref[...] = acc_ref[...].astype(jnp.bfloat16)


def _pool(x_grouped, wp, kb):
    m, kdim = x_grouped.shape
    n_k = kdim // kb
    return pl.pallas_call(
        functools.partial(_pool_kernel, n_k),
        grid=(n_k,),
        in_specs=[
            pl.BlockSpec((m, kb), lambda k: (0, k)),
            pl.BlockSpec((kb, D_MODEL), lambda k: (k, 0)),
        ],
        out_specs=pl.BlockSpec((m, D_MODEL), lambda k: (0, 0)),
        out_shape=jax.ShapeDtypeStruct((m, D_MODEL), jnp.bfloat16),
        scratch_shapes=[pltpu.VMEM((m, D_MODEL), jnp.float32)],
    )(x_grouped, wp)


def _ffn_kernel(n_blocks, nb, r_ref, wg_ref, wu_ref, wd_ref, o_ref, acc_ref):
    n = pl.program_id(1)
    lim = D_FFN - n * nb
    col = jax.lax.broadcasted_iota(jnp.int32, (D_MODEL, nb), 1)
    row = jax.lax.broadcasted_iota(jnp.int32, (nb, D_MODEL), 0)
    wg = jnp.where(col < lim, wg_ref[0, 0], 0.0).astype(jnp.bfloat16)
    wu = jnp.where(col < lim, wu_ref[0, 0], 0.0).astype(jnp.bfloat16)
    wd = jnp.where(row < lim, wd_ref[0, 0], 0.0).astype(jnp.bfloat16)

    r = r_ref[...]
    a = jnp.dot(r, wg, preferred_element_type=jnp.float32)
    b = jnp.dot(r, wu, preferred_element_type=jnp.float32)
    h = (a * jax.nn.sigmoid(a) * b).astype(jnp.bfloat16)
    part = jnp.dot(h, wd, preferred_element_type=jnp.float32)

    @pl.when(n == 0)
    def _():
        acc_ref[...] = jnp.zeros_like(acc_ref)

    acc_ref[...] += part

    @pl.when(n == n_blocks - 1)
    def _():
        o_ref[0] = acc_ref[...].astype(jnp.bfloat16)


def _ffn_experts(r, wg, wu, wd, scale_idx, nb):
    m = r.shape[0]
    n_blocks = (D_FFN + nb - 1) // nb
    return pl.pallas_call(
        functools.partial(_ffn_kernel, n_blocks, nb),
        grid=(N_EXP, n_blocks),
        in_specs=[
            pl.BlockSpec((m, D_MODEL), lambda e, n: (0, 0)),
            pl.BlockSpec((1, 1, D_MODEL, nb), lambda e, n: (scale_idx, e, 0, n)),
            pl.BlockSpec((1, 1, D_MODEL, nb), lambda e, n: (scale_idx, e, 0, n)),
            pl.BlockSpec((1, 1, nb, D_MODEL), lambda e, n: (scale_idx, e, n, 0)),
        ],
        out_specs=pl.BlockSpec((1, m, D_MODEL), lambda e, n: (e, 0, 0)),
        out_shape=jax.ShapeDtypeStruct((N_EXP, m, D_MODEL), jnp.bfloat16),
        scratch_shapes=[pltpu.VMEM((m, D_MODEL), jnp.float32)],
    )(r, wg, wu, wd)


def _ffn0_kernel(n_blocks, nb, mb, n_tok, x_ref, w_ref, wg_ref, wu_ref, wd_ref,
                 e1_ref, e2_ref, part_ref, o_ref, aux_ref):
    i = pl.program_id(0)
    e = pl.program_id(1)
    n = pl.program_id(2)

    @pl.when(jnp.logical_and(i == 0, jnp.logical_and(e == 0, n == 0)))
    def _():
        p = jnp.sum(part_ref[...], axis=0)
        r192 = jax.lax.broadcasted_iota(jnp.int32, (N_TOTAL * 16, N_TOTAL), 0)
        c192 = jax.lax.broadcasted_iota(jnp.int32, (N_TOTAL * 16, N_TOTAL), 1)
        sel = (r192 // 16 == c192).astype(jnp.float32)
        sums = jnp.dot(p, sel, preferred_element_type=jnp.float32)
        load = sums[0:1, :] / n_tok
        frac = sums[1:2, :] / (n_tok * TOP_K)
        aux_ref[...] = (N_TOTAL * jnp.sum(frac * load)).reshape(1, 1)

    wt = w_ref[...]
    lim = D_FFN - n * nb
    col = jax.lax.broadcasted_iota(jnp.int32, (D_MODEL, nb), 1)
    row = jax.lax.broadcasted_iota(jnp.int32, (nb, D_MODEL), 0)
    wg = jnp.where(col < lim, wg_ref[0, 0], 0.0).astype(jnp.bfloat16)
    wu = jnp.where(col < lim, wu_ref[0, 0], 0.0).astype(jnp.bfloat16)
    wd = jnp.where(row < lim, wd_ref[0, 0], 0.0).astype(jnp.bfloat16)

    x = x_ref[...]
    sub = jax.lax.broadcasted_iota(jnp.int32, wt.shape, 0)
    wcol = jnp.sum(jnp.where(sub == e, wt, 0.0), axis=0, keepdims=True).T
    a = jnp.dot(x, wg, preferred_element_type=jnp.float32)
    b = jnp.dot(x, wu, preferred_element_type=jnp.float32)
    h = (a * jax.nn.sigmoid(a) * b * wcol).astype(jnp.bfloat16)
    part = jnp.dot(h, wd, preferred_element_type=jnp.float32)

    @pl.when(jnp.logical_and(e == 0, n == 0))
    def _():
        g1, g2 = mb // 4, mb // 16
        acc = jnp.zeros((mb, D_MODEL), jnp.float32)
        for ee in range(N_EXP):
            v1 = e1_ref[ee].astype(jnp.float32)
            v1r = jnp.broadcast_to(v1[:, None, :], (g1, 4, D_MODEL))
            v1r = v1r.reshape(mb, D_MODEL)
            acc += wt[N_EXP + ee:N_EXP + ee + 1, :].T * v1r
            v2 = e2_ref[ee].astype(jnp.float32)
            v2r = jnp.broadcast_to(v2[:, None, :], (g2, 16, D_MODEL))
            v2r = v2r.reshape(mb, D_MODEL)
            acc += wt[2 * N_EXP + ee:2 * N_EXP + ee + 1, :].T * v2r
        o_ref[...] = acc

    o_ref[...] += part


def _ffn0(xb, w, wg, wu, wd, e1, e2, part, nb, mb):
    m = xb.shape[0]
    n_blocks = (D_FFN + nb - 1) // nb
    return pl.pallas_call(
        functools.partial(_ffn0_kernel, n_blocks, nb, mb, m),
        grid=(m // mb, N_EXP, n_blocks),
        in_specs=[
            pl.BlockSpec((mb, D_MODEL), lambda i, e, n: (i, 0)),
            pl.BlockSpec((N_TOTAL, mb), lambda i, e, n: (0, i)),
            pl.BlockSpec((1, 1, D_MODEL, nb), lambda i, e, n: (0, e, 0, n)),
            pl.BlockSpec((1, 1, D_MODEL, nb), lambda i, e, n: (0, e, 0, n)),
            pl.BlockSpec((1, 1, nb, D_MODEL), lambda i, e, n: (0, e, n, 0)),
            pl.BlockSpec((N_EXP, mb // 4, D_MODEL), lambda i, e, n: (0, i, 0)),
            pl.BlockSpec((N_EXP, mb // 16, D_MODEL), lambda i, e, n: (0, i, 0)),
            pl.BlockSpec((_SC_NW, 2, N_TOTAL * 16), lambda i, e, n: (0, 0, 0)),
        ],
        out_specs=[
            pl.BlockSpec((mb, D_MODEL), lambda i, e, n: (i, 0)),
            pl.BlockSpec((1, 1), lambda i, e, n: (0, 0)),
        ],
        out_shape=[
            jax.ShapeDtypeStruct((m, D_MODEL), jnp.float32),
            jax.ShapeDtypeStruct((1, 1), jnp.float32),
        ],
    )(xb, w, wg, wu, wd, e1, e2, part)


def kernel(x, Wr, Wp1, Wp2, Wg, Wu, Wd):
    B, T, D = x.shape
    n_tok = B * T
    x_flat = x.reshape(n_tok, D)
    x1 = x.reshape(n_tok // 4, 4 * D)
    x2 = x.reshape(n_tok // 16, 16 * D)

    ltr = _logits(x_flat, Wr)
    wt, part = _sc_router(ltr)
    r1 = _pool(x1, Wp1, 1024)
    r2 = _pool(x2, Wp2, 2048)
    e1 = _ffn_experts(r1, Wg, Wu, Wd, 1, 512)
    e2 = _ffn_experts(r2, Wg, Wu, Wd, 2, 512)
    out, aux = _ffn0(x_flat.astype(jnp.bfloat16), wt, Wg, Wu, Wd, e1, e2, part,
                     512, 1024)
    return out.reshape(B, T, D), aux.reshape(())


# speedup vs baseline: 1.0012x; 1.0012x over previous
import functools

import jax
import jax.numpy as jnp
from jax import lax
from jax.experimental import pallas as pl
from jax.experimental.pallas import tpu as pltpu
from jax.experimental.pallas import tpu_sc as plsc

D_MODEL = 1024
N_SCALES = 3
N_EXP = 4
N_TOTAL = 12
TOP_K = 2
D_FFN = 2752


def _logits_kernel(x_ref, wr_ref, lt_ref):
    logits = jnp.dot(x_ref[...], wr_ref[...],
                     preferred_element_type=jnp.float32)
    lt_ref[...] = logits.T


def _logits(x_flat, wr):
    n_tok = x_flat.shape[0]
    blk = 2048
    return pl.pallas_call(
        _logits_kernel,
        grid=(n_tok // blk,),
        in_specs=[
            pl.BlockSpec((blk, D_MODEL), lambda i: (i, 0)),
            pl.BlockSpec((D_MODEL, N_TOTAL), lambda i: (0, 0)),
        ],
        out_specs=pl.BlockSpec((N_TOTAL, blk), lambda i: (0, i)),
        out_shape=jax.ShapeDtypeStruct((N_TOTAL, n_tok), jnp.float32),
    )(x_flat, wr)


_SC_NW = 32
_SC_TPW = 128
_SC_G = _SC_TPW // 16


def _sc_router_body(lt_hbm, w_hbm, part_hbm, lbuf, wbuf, pbuf):
    c = lax.axis_index("c")
    s = lax.axis_index("s")
    wid = s * 2 + c
    t0 = pl.multiple_of(wid * _SC_TPW, _SC_TPW)
    pltpu.sync_copy(lt_hbm.at[:, pl.ds(t0, _SC_TPW)], lbuf)

    iota = lax.iota(jnp.int32, 16)
    acc_load = [jnp.zeros((16,), jnp.float32) for _ in range(N_TOTAL)]
    acc_cnt = [jnp.zeros((16,), jnp.float32) for _ in range(N_TOTAL)]
    for g in range(_SC_G):
        ls = [lbuf[e, pl.ds(g * 16, 16)] for e in range(N_TOTAL)]
        m = ls[0]
        for e in range(1, N_TOTAL):
            m = jnp.maximum(m, ls[e])
        es = [jnp.exp(ls[e] - m) for e in range(N_TOTAL)]
        z = es[0]
        for e in range(1, N_TOTAL):
            z = z + es[e]
        zi = 1.0 / z
        e1 = es[0]
        i1 = jnp.zeros((16,), jnp.int32)
        e2 = jnp.zeros((16,), jnp.float32)
        i2 = jnp.full((16,), N_TOTAL, jnp.int32)
        for e in range(1, N_TOTAL):
            v = es[e]
            gt1 = v > e1
            gt2 = v > e2
            i2 = jnp.where(gt1, i1, jnp.where(gt2, e, i2))
            e2 = jnp.where(gt1, e1, jnp.where(gt2, v, e2))
            i1 = jnp.where(gt1, e, i1)
            e1 = jnp.where(gt1, v, e1)
        d = e1 + e2 + 1e-8 * z
        w1 = e1 / d
        w2 = e2 / d
        for e in range(N_TOTAL):
            we = (jnp.where(i1 == e, w1, 0.0) + jnp.where(i2 == e, w2, 0.0))
            wbuf[e, pl.ds(g * 16, 16)] = we
            acc_load[e] = acc_load[e] + es[e] * zi
            acc_cnt[e] = (acc_cnt[e]
                          + jnp.where(i1 == e, 1.0, 0.0)
                          + jnp.where(i2 == e, 1.0, 0.0))

    for e in range(N_TOTAL):
        pbuf[0, pl.ds(e * 16, 16)] = acc_load[e]
        pbuf[1, pl.ds(e * 16, 16)] = acc_cnt[e]
    pltpu.sync_copy(wbuf, w_hbm.at[:, pl.ds(t0, _SC_TPW)])
    pltpu.sync_copy(pbuf, part_hbm.at[wid])


def _sc_router(ltr):
    n_tok = ltr.shape[1]
    return pl.kernel(
        _sc_router_body,
        out_type=[
            jax.ShapeDtypeStruct((N_TOTAL, n_tok), jnp.float32),
            jax.ShapeDtypeStruct((_SC_NW, 2, N_TOTAL * 16), jnp.float32),
        ],
        mesh=plsc.VectorSubcoreMesh(core_axis_name="c", subcore_axis_name="s",
                                    num_cores=2, num_subcores=16),
        scratch_types=[
            pltpu.VMEM((N_TOTAL, _SC_TPW), jnp.float32),
            pltpu.VMEM((N_TOTAL, _SC_TPW), jnp.float32),
            pltpu.VMEM((2, N_TOTAL * 16), jnp.float32),
        ],
    )(ltr)


def _pools_kernel(x1_ref, wp1_ref, x2_ref, wp2_ref, o1_ref, o2_ref,
                  acc1_ref, acc2_ref):
    k = pl.program_id(0)

    @pl.when(k == 0)
    def _():
        acc1_ref[...] = jnp.zeros_like(acc1_ref)
        acc2_ref[...] = jnp.zeros_like(acc2_ref)

    @pl.when(k < 4)
    def _():
        xb = x1_ref[...].astype(jnp.bfloat16)
        wb = wp1_ref[...].astype(jnp.bfloat16)
        acc1_ref[...] += jnp.dot(xb, wb, preferred_element_type=jnp.float32)

    @pl.when(k == 3)
    def _():
        o1_ref[...] = acc1_ref[...].astype(jnp.bfloat16)

    @pl.when(k >= 4)
    def _():
        xb = x2_ref[...].astype(jnp.bfloat16)
        wb = wp2_ref[...].astype(jnp.bfloat16)
        acc2_ref[...] += jnp.dot(xb, wb, preferred_element_type=jnp.float32)

    @pl.when(k == 19)
    def _():
        o2_ref[...] = acc2_ref[...].astype(jnp.bfloat16)


def _pools(x1, wp1, x2, wp2):
    m1 = x1.shape[0]
    m2 = x2.shape[0]
    return pl.pallas_call(
        _pools_kernel,
        grid=(20,),
        in_specs=[
            pl.BlockSpec((m1, 1024), lambda k: (0, jnp.minimum(k, 3))),
            pl.BlockSpec((1024, D_MODEL), lambda k: (jnp.minimum(k, 3), 0)),
            pl.BlockSpec((m2, 1024), lambda k: (0, jnp.maximum(k - 4, 0))),
            pl.BlockSpec((1024, D_MODEL), lambda k: (jnp.maximum(k - 4, 0), 0)),
        ],
        out_specs=[
            pl.BlockSpec((m1, D_MODEL), lambda k: (0, 0)),
            pl.BlockSpec((m2, D_MODEL), lambda k: (0, 0)),
        ],
        out_shape=[
            jax.ShapeDtypeStruct((m1, D_MODEL), jnp.bfloat16),
            jax.ShapeDtypeStruct((m2, D_MODEL), jnp.bfloat16),
        ],
        scratch_shapes=[pltpu.VMEM((m1, D_MODEL), jnp.float32),
                        pltpu.VMEM((m2, D_MODEL), jnp.float32)],
    )(x1, wp1, x2, wp2)


def _ffn_kernel(n_blocks, nb, r_ref, wg_ref, wu_ref, wd_ref, o_ref, acc_ref):
    n = pl.program_id(1)
    lim = D_FFN - n * nb
    col = jax.lax.broadcasted_iota(jnp.int32, (D_MODEL, nb), 1)
    row = jax.lax.broadcasted_iota(jnp.int32, (nb, D_MODEL), 0)
    wg = jnp.where(col < lim, wg_ref[0, 0], 0.0).astype(jnp.bfloat16)
    wu = jnp.where(col < lim, wu_ref[0, 0], 0.0).astype(jnp.bfloat16)
    wd = jnp.where(row < lim, wd_ref[0, 0], 0.0).astype(jnp.bfloat16)

    r = r_ref[...]
    a = jnp.dot(r, wg, preferred_element_type=jnp.float32)
    b = jnp.dot(r, wu, preferred_element_type=jnp.float32)
    h = (a * jax.nn.sigmoid(a) * b).astype(jnp.bfloat16)
    part = jnp.dot(h, wd, preferred_element_type=jnp.float32)

    @pl.when(n == 0)
    def _():
        acc_ref[...] = jnp.zeros_like(acc_ref)

    acc_ref[...] += part

    @pl.when(n == n_blocks - 1)
    def _():
        o_ref[0] = acc_ref[...].astype(jnp.bfloat16)


def _ffn_experts(r, wg, wu, wd, scale_idx, nb):
    m = r.shape[0]
    n_blocks = (D_FFN + nb - 1) // nb
    return pl.pallas_call(
        functools.partial(_ffn_kernel, n_blocks, nb),
        grid=(N_EXP, n_blocks),
        in_specs=[
            pl.BlockSpec((m, D_MODEL), lambda e, n: (0, 0)),
            pl.BlockSpec((1, 1, D_MODEL, nb), lambda e, n: (scale_idx, e, 0, n)),
            pl.BlockSpec((1, 1, D_MODEL, nb), lambda e, n: (scale_idx, e, 0, n)),
            pl.BlockSpec((1, 1, nb, D_MODEL), lambda e, n: (scale_idx, e, n, 0)),
        ],
        out_specs=pl.BlockSpec((1, m, D_MODEL), lambda e, n: (e, 0, 0)),
        out_shape=jax.ShapeDtypeStruct((N_EXP, m, D_MODEL), jnp.bfloat16),
        scratch_shapes=[pltpu.VMEM((m, D_MODEL), jnp.float32)],
    )(r, wg, wu, wd)


def _ffn0_kernel(n_blocks, nb, mb, n_tok, x_ref, w_ref, wg_ref, wu_ref, wd_ref,
                 e1_ref, e2_ref, part_ref, o_ref, aux_ref, wtok_ref):
    i = pl.program_id(0)
    e = pl.program_id(1)
    n = pl.program_id(2)

    @pl.when(jnp.logical_and(i == 0, jnp.logical_and(e == 0, n == 0)))
    def _():
        p = jnp.sum(part_ref[...], axis=0)
        r192 = jax.lax.broadcasted_iota(jnp.int32, (N_TOTAL * 16, N_TOTAL), 0)
        c192 = jax.lax.broadcasted_iota(jnp.int32, (N_TOTAL * 16, N_TOTAL), 1)
        sel = (r192 // 16 == c192).astype(jnp.float32)
        sums = jnp.dot(p, sel, preferred_element_type=jnp.float32)
        load = sums[0:1, :] / n_tok
        frac = sums[1:2, :] / (n_tok * TOP_K)
        aux_ref[...] = (N_TOTAL * jnp.sum(frac * load)).reshape(1, 1)

    @pl.when(jnp.logical_and(e == 0, n == 0))
    def _():
        wtok_ref[...] = w_ref[...].T

    lim = D_FFN - n * nb
    col = jax.lax.broadcasted_iota(jnp.int32, (D_MODEL, nb), 1)
    row = jax.lax.broadcasted_iota(jnp.int32, (nb, D_MODEL), 0)
    wg = jnp.where(col < lim, wg_ref[0, 0], 0.0).astype(jnp.bfloat16)
    wu = jnp.where(col < lim, wu_ref[0, 0], 0.0).astype(jnp.bfloat16)
    wd = jnp.where(row < lim, wd_ref[0, 0], 0.0).astype(jnp.bfloat16)

    x = x_ref[...]
    wtok = wtok_ref[...]
    lane = jax.lax.broadcasted_iota(jnp.int32, wtok.shape, 1)
    wcol = jnp.sum(jnp.where(lane == e, wtok, 0.0), axis=1, keepdims=True)
    a = jnp.dot(x, wg, preferred_element_type=jnp.float32)
    b = jnp.dot(x, wu, preferred_element_type=jnp.float32)
    h = (a * jax.nn.sigmoid(a) * b * wcol).astype(jnp.bfloat16)
    part = jnp.dot(h, wd, preferred_element_type=jnp.float32)

    @pl.when(jnp.logical_and(e == 0, n == 0))
    def _():
        g1, g2 = mb // 4, mb // 16
        wtok = wtok_ref[...]
        acc = jnp.zeros((mb, D_MODEL), jnp.float32)
        for ee in range(N_EXP):
            v1 = e1_ref[ee].astype(jnp.float32)
            v1r = jnp.broadcast_to(v1[:, None, :], (g1, 4, D_MODEL))
            v1r = v1r.reshape(mb, D_MODEL)
            acc += wtok[:, N_EXP + ee:N_EXP + ee + 1] * v1r
            v2 = e2_ref[ee].astype(jnp.float32)
            v2r = jnp.broadcast_to(v2[:, None, :], (g2, 16, D_MODEL))
            v2r = v2r.reshape(mb, D_MODEL)
            acc += wtok[:, 2 * N_EXP + ee:2 * N_EXP + ee + 1] * v2r
        o_ref[...] = acc

    o_ref[...] += part


def _ffn0(xb, w, wg, wu, wd, e1, e2, part, nb, mb):
    m = xb.shape[0]
    n_blocks = (D_FFN + nb - 1) // nb
    return pl.pallas_call(
        functools.partial(_ffn0_kernel, n_blocks, nb, mb, m),
        grid=(m // mb, N_EXP, n_blocks),
        in_specs=[
            pl.BlockSpec((mb, D_MODEL), lambda i, e, n: (i, 0)),
            pl.BlockSpec((N_TOTAL, mb), lambda i, e, n: (0, i)),
            pl.BlockSpec((1, 1, D_MODEL, nb), lambda i, e, n: (0, e, 0, n)),
            pl.BlockSpec((1, 1, D_MODEL, nb), lambda i, e, n: (0, e, 0, n)),
            pl.BlockSpec((1, 1, nb, D_MODEL), lambda i, e, n: (0, e, n, 0)),
            pl.BlockSpec((N_EXP, mb // 4, D_MODEL), lambda i, e, n: (0, i, 0)),
            pl.BlockSpec((N_EXP, mb // 16, D_MODEL), lambda i, e, n: (0, i, 0)),
            pl.BlockSpec((_SC_NW, 2, N_TOTAL * 16), lambda i, e, n: (0, 0, 0)),
        ],
        out_specs=[
            pl.BlockSpec((mb, D_MODEL), lambda i, e, n: (i, 0)),
            pl.BlockSpec((1, 1), lambda i, e, n: (0, 0)),
        ],
        out_shape=[
            jax.ShapeDtypeStruct((m, D_MODEL), jnp.float32),
            jax.ShapeDtypeStruct((1, 1), jnp.float32),
        ],
        scratch_shapes=[pltpu.VMEM((mb, N_TOTAL), jnp.float32)],
    )(xb, w, wg, wu, wd, e1, e2, part)


def kernel(x, Wr, Wp1, Wp2, Wg, Wu, Wd):
    B, T, D = x.shape
    n_tok = B * T
    x_flat = x.reshape(n_tok, D)
    x1 = x.reshape(n_tok // 4, 4 * D)
    x2 = x.reshape(n_tok // 16, 16 * D)

    ltr = _logits(x_flat, Wr)
    wt, part = _sc_router(ltr)
    r1, r2 = _pools(x1, Wp1, x2, Wp2)
    e1 = _ffn_experts(r1, Wg, Wu, Wd, 1, 512)
    e2 = _ffn_experts(r2, Wg, Wu, Wd, 2, 512)
    out, aux = _ffn0(x_flat.astype(jnp.bfloat16), wt, Wg, Wu, Wd, e1, e2, part,
                     512, 1024)
    return out.reshape(B, T, D), aux.reshape(())


# speedup vs baseline: 1.0019x; 1.0007x over previous
import functools

import jax
import jax.numpy as jnp
from jax import lax
from jax.experimental import pallas as pl
from jax.experimental.pallas import tpu as pltpu
from jax.experimental.pallas import tpu_sc as plsc

D_MODEL = 1024
N_SCALES = 3
N_EXP = 4
N_TOTAL = 12
TOP_K = 2
D_FFN = 2752


def _logits_kernel(x_ref, wr_ref, lt_ref):
    logits = jnp.dot(x_ref[...], wr_ref[...],
                     preferred_element_type=jnp.float32)
    lt_ref[...] = logits.T


def _logits(x_flat, wr):
    n_tok = x_flat.shape[0]
    blk = 2048
    return pl.pallas_call(
        _logits_kernel,
        grid=(n_tok // blk,),
        in_specs=[
            pl.BlockSpec((blk, D_MODEL), lambda i: (i, 0)),
            pl.BlockSpec((D_MODEL, N_TOTAL), lambda i: (0, 0)),
        ],
        out_specs=pl.BlockSpec((N_TOTAL, blk), lambda i: (0, i)),
        out_shape=jax.ShapeDtypeStruct((N_TOTAL, n_tok), jnp.float32),
    )(x_flat, wr)


_SC_NW = 32
_SC_TPW = 128
_SC_G = _SC_TPW // 16


def _sc_router_body(lt_hbm, w_hbm, part_hbm, lbuf, wbuf, pbuf):
    c = lax.axis_index("c")
    s = lax.axis_index("s")
    wid = s * 2 + c
    t0 = pl.multiple_of(wid * _SC_TPW, _SC_TPW)
    pltpu.sync_copy(lt_hbm.at[:, pl.ds(t0, _SC_TPW)], lbuf)

    iota = lax.iota(jnp.int32, 16)
    acc_load = [jnp.zeros((16,), jnp.float32) for _ in range(N_TOTAL)]
    acc_cnt = [jnp.zeros((16,), jnp.float32) for _ in range(N_TOTAL)]
    for g in range(_SC_G):
        ls = [lbuf[e, pl.ds(g * 16, 16)] for e in range(N_TOTAL)]
        m = ls[0]
        for e in range(1, N_TOTAL):
            m = jnp.maximum(m, ls[e])
        es = [jnp.exp(ls[e] - m) for e in range(N_TOTAL)]
        z = es[0]
        for e in range(1, N_TOTAL):
            z = z + es[e]
        zi = 1.0 / z
        e1 = es[0]
        i1 = jnp.zeros((16,), jnp.int32)
        e2 = jnp.zeros((16,), jnp.float32)
        i2 = jnp.full((16,), N_TOTAL, jnp.int32)
        for e in range(1, N_TOTAL):
            v = es[e]
            gt1 = v > e1
            gt2 = v > e2
            i2 = jnp.where(gt1, i1, jnp.where(gt2, e, i2))
            e2 = jnp.where(gt1, e1, jnp.where(gt2, v, e2))
            i1 = jnp.where(gt1, e, i1)
            e1 = jnp.where(gt1, v, e1)
        d = e1 + e2 + 1e-8 * z
        w1 = e1 / d
        w2 = e2 / d
        for e in range(N_TOTAL):
            we = (jnp.where(i1 == e, w1, 0.0) + jnp.where(i2 == e, w2, 0.0))
            wbuf[e, pl.ds(g * 16, 16)] = we
            acc_load[e] = acc_load[e] + es[e] * zi
            acc_cnt[e] = (acc_cnt[e]
                          + jnp.where(i1 == e, 1.0, 0.0)
                          + jnp.where(i2 == e, 1.0, 0.0))

    for e in range(N_TOTAL):
        pbuf[0, pl.ds(e * 16, 16)] = acc_load[e]
        pbuf[1, pl.ds(e * 16, 16)] = acc_cnt[e]
    pltpu.sync_copy(wbuf, w_hbm.at[:, pl.ds(t0, _SC_TPW)])
    pltpu.sync_copy(pbuf, part_hbm.at[wid])


def _sc_router(ltr):
    n_tok = ltr.shape[1]
    return pl.kernel(
        _sc_router_body,
        out_type=[
            jax.ShapeDtypeStruct((N_TOTAL, n_tok), jnp.float32),
            jax.ShapeDtypeStruct((_SC_NW, 2, N_TOTAL * 16), jnp.float32),
        ],
        mesh=plsc.VectorSubcoreMesh(core_axis_name="c", subcore_axis_name="s",
                                    num_cores=2, num_subcores=16),
        scratch_types=[
            pltpu.VMEM((N_TOTAL, _SC_TPW), jnp.float32),
            pltpu.VMEM((N_TOTAL, _SC_TPW), jnp.float32),
            pltpu.VMEM((2, N_TOTAL * 16), jnp.float32),
        ],
    )(ltr)


def _pools_kernel(x1_ref, wp1_ref, x2_ref, wp2_ref, o1_ref, o2_ref,
                  acc1_ref, acc2_ref):
    k = pl.program_id(0)

    @pl.when(k == 0)
    def _():
        acc1_ref[...] = jnp.zeros_like(acc1_ref)
        acc2_ref[...] = jnp.zeros_like(acc2_ref)

    @pl.when(k < 4)
    def _():
        xb = x1_ref[...].astype(jnp.bfloat16)
        wb = wp1_ref[...].astype(jnp.bfloat16)
        acc1_ref[...] += jnp.dot(xb, wb, preferred_element_type=jnp.float32)

    @pl.when(k == 3)
    def _():
        o1_ref[...] = acc1_ref[...].astype(jnp.bfloat16)

    @pl.when(k >= 4)
    def _():
        xb = x2_ref[...].astype(jnp.bfloat16)
        wb = wp2_ref[...].astype(jnp.bfloat16)
        acc2_ref[...] += jnp.dot(xb, wb, preferred_element_type=jnp.float32)

    @pl.when(k == 19)
    def _():
        o2_ref[...] = acc2_ref[...].astype(jnp.bfloat16)


def _pools(x1, wp1, x2, wp2):
    m1 = x1.shape[0]
    m2 = x2.shape[0]
    return pl.pallas_call(
        _pools_kernel,
        grid=(20,),
        in_specs=[
            pl.BlockSpec((m1, 1024), lambda k: (0, jnp.minimum(k, 3))),
            pl.BlockSpec((1024, D_MODEL), lambda k: (jnp.minimum(k, 3), 0)),
            pl.BlockSpec((m2, 1024), lambda k: (0, jnp.maximum(k - 4, 0))),
            pl.BlockSpec((1024, D_MODEL), lambda k: (jnp.maximum(k - 4, 0), 0)),
        ],
        out_specs=[
            pl.BlockSpec((m1, D_MODEL), lambda k: (0, 0)),
            pl.BlockSpec((m2, D_MODEL), lambda k: (0, 0)),
        ],
        out_shape=[
            jax.ShapeDtypeStruct((m1, D_MODEL), jnp.bfloat16),
            jax.ShapeDtypeStruct((m2, D_MODEL), jnp.bfloat16),
        ],
        scratch_shapes=[pltpu.VMEM((m1, D_MODEL), jnp.float32),
                        pltpu.VMEM((m2, D_MODEL), jnp.float32)],
    )(x1, wp1, x2, wp2)


def _masked_weights(nb, n, wg_ref, wu_ref, wd_ref):
    if D_FFN % nb == 0:
        return (wg_ref[0, 0].astype(jnp.bfloat16),
                wu_ref[0, 0].astype(jnp.bfloat16),
                wd_ref[0, 0].astype(jnp.bfloat16))
    lim = D_FFN - n * nb
    col = jax.lax.broadcasted_iota(jnp.int32, (D_MODEL, nb), 1)
    row = jax.lax.broadcasted_iota(jnp.int32, (nb, D_MODEL), 0)
    return (jnp.where(col < lim, wg_ref[0, 0], 0.0).astype(jnp.bfloat16),
            jnp.where(col < lim, wu_ref[0, 0], 0.0).astype(jnp.bfloat16),
            jnp.where(row < lim, wd_ref[0, 0], 0.0).astype(jnp.bfloat16))


def _ffn_kernel(n_blocks, nb, r_ref, wg_ref, wu_ref, wd_ref, o_ref, acc_ref):
    n = pl.program_id(1)
    wg, wu, wd = _masked_weights(nb, n, wg_ref, wu_ref, wd_ref)

    r = r_ref[...]
    a = jnp.dot(r, wg, preferred_element_type=jnp.float32)
    b = jnp.dot(r, wu, preferred_element_type=jnp.float32)
    h = (a * jax.nn.sigmoid(a) * b).astype(jnp.bfloat16)
    part = jnp.dot(h, wd, preferred_element_type=jnp.float32)

    @pl.when(n == 0)
    def _():
        acc_ref[...] = jnp.zeros_like(acc_ref)

    acc_ref[...] += part

    @pl.when(n == n_blocks - 1)
    def _():
        o_ref[0] = acc_ref[...].astype(jnp.bfloat16)


def _ffn_experts(r, wg, wu, wd, scale_idx, nb):
    m = r.shape[0]
    n_blocks = (D_FFN + nb - 1) // nb
    return pl.pallas_call(
        functools.partial(_ffn_kernel, n_blocks, nb),
        grid=(N_EXP, n_blocks),
        in_specs=[
            pl.BlockSpec((m, D_MODEL), lambda e, n: (0, 0)),
            pl.BlockSpec((1, 1, D_MODEL, nb), lambda e, n: (scale_idx, e, 0, n)),
            pl.BlockSpec((1, 1, D_MODEL, nb), lambda e, n: (scale_idx, e, 0, n)),
            pl.BlockSpec((1, 1, nb, D_MODEL), lambda e, n: (scale_idx, e, n, 0)),
        ],
        out_specs=pl.BlockSpec((1, m, D_MODEL), lambda e, n: (e, 0, 0)),
        out_shape=jax.ShapeDtypeStruct((N_EXP, m, D_MODEL), jnp.bfloat16),
        scratch_shapes=[pltpu.VMEM((m, D_MODEL), jnp.float32)],
    )(r, wg, wu, wd)


def _ffn0_kernel(n_blocks, nb, mb, n_tok, x_ref, w_ref, wg_ref, wu_ref, wd_ref,
                 e1_ref, e2_ref, part_ref, o_ref, aux_ref, wtok_ref):
    i = pl.program_id(0)
    e = pl.program_id(1)
    n = pl.program_id(2)

    @pl.when(jnp.logical_and(i == 0, jnp.logical_and(e == 0, n == 0)))
    def _():
        p = jnp.sum(part_ref[...], axis=0)
        r192 = jax.lax.broadcasted_iota(jnp.int32, (N_TOTAL * 16, N_TOTAL), 0)
        c192 = jax.lax.broadcasted_iota(jnp.int32, (N_TOTAL * 16, N_TOTAL), 1)
        sel = (r192 // 16 == c192).astype(jnp.float32)
        sums = jnp.dot(p, sel, preferred_element_type=jnp.float32)
        load = sums[0:1, :] / n_tok
        frac = sums[1:2, :] / (n_tok * TOP_K)
        aux_ref[...] = (N_TOTAL * jnp.sum(frac * load)).reshape(1, 1)

    @pl.when(jnp.logical_and(e == 0, n == 0))
    def _():
        wtok_ref[...] = w_ref[...].T

    wg, wu, wd = _masked_weights(nb, n, wg_ref, wu_ref, wd_ref)

    x = x_ref[...]
    wtok = wtok_ref[...]
    lane = jax.lax.broadcasted_iota(jnp.int32, wtok.shape, 1)
    wcol = jnp.sum(jnp.where(lane == e, wtok, 0.0), axis=1, keepdims=True)
    a = jnp.dot(x, wg, preferred_element_type=jnp.float32)
    b = jnp.dot(x, wu, preferred_element_type=jnp.float32)
    h = (a * jax.nn.sigmoid(a) * b * wcol).astype(jnp.bfloat16)
    part = jnp.dot(h, wd, preferred_element_type=jnp.float32)

    @pl.when(jnp.logical_and(e == 0, n == 0))
    def _():
        g1, g2 = mb // 4, mb // 16
        wtok = wtok_ref[...]
        acc = jnp.zeros((mb, D_MODEL), jnp.float32)
        for ee in range(N_EXP):
            v1 = e1_ref[ee].astype(jnp.float32)
            v1r = jnp.broadcast_to(v1[:, None, :], (g1, 4, D_MODEL))
            v1r = v1r.reshape(mb, D_MODEL)
            acc += wtok[:, N_EXP + ee:N_EXP + ee + 1] * v1r
            v2 = e2_ref[ee].astype(jnp.float32)
            v2r = jnp.broadcast_to(v2[:, None, :], (g2, 16, D_MODEL))
            v2r = v2r.reshape(mb, D_MODEL)
            acc += wtok[:, 2 * N_EXP + ee:2 * N_EXP + ee + 1] * v2r
        o_ref[...] = acc

    o_ref[...] += part


def _ffn0(xb, w, wg, wu, wd, e1, e2, part, nb, mb):
    m = xb.shape[0]
    n_blocks = (D_FFN + nb - 1) // nb
    return pl.pallas_call(
        functools.partial(_ffn0_kernel, n_blocks, nb, mb, m),
        grid=(m // mb, N_EXP, n_blocks),
        in_specs=[
            pl.BlockSpec((mb, D_MODEL), lambda i, e, n: (i, 0)),
            pl.BlockSpec((N_TOTAL, mb), lambda i, e, n: (0, i)),
            pl.BlockSpec((1, 1, D_MODEL, nb), lambda i, e, n: (0, e, 0, n)),
            pl.BlockSpec((1, 1, D_MODEL, nb), lambda i, e, n: (0, e, 0, n)),
            pl.BlockSpec((1, 1, nb, D_MODEL), lambda i, e, n: (0, e, n, 0)),
            pl.BlockSpec((N_EXP, mb // 4, D_MODEL), lambda i, e, n: (0, i, 0)),
            pl.BlockSpec((N_EXP, mb // 16, D_MODEL), lambda i, e, n: (0, i, 0)),
            pl.BlockSpec((_SC_NW, 2, N_TOTAL * 16), lambda i, e, n: (0, 0, 0)),
        ],
        out_specs=[
            pl.BlockSpec((mb, D_MODEL), lambda i, e, n: (i, 0)),
            pl.BlockSpec((1, 1), lambda i, e, n: (0, 0)),
        ],
        out_shape=[
            jax.ShapeDtypeStruct((m, D_MODEL), jnp.float32),
            jax.ShapeDtypeStruct((1, 1), jnp.float32),
        ],
        scratch_shapes=[pltpu.VMEM((mb, N_TOTAL), jnp.float32)],
    )(xb, w, wg, wu, wd, e1, e2, part)


def kernel(x, Wr, Wp1, Wp2, Wg, Wu, Wd):
    B, T, D = x.shape
    n_tok = B * T
    x_flat = x.reshape(n_tok, D)
    x1 = x.reshape(n_tok // 4, 4 * D)
    x2 = x.reshape(n_tok // 16, 16 * D)

    ltr = _logits(x_flat, Wr)
    wt, part = _sc_router(ltr)
    r1, r2 = _pools(x1, Wp1, x2, Wp2)
    e1 = _ffn_experts(r1, Wg, Wu, Wd, 1, 512)
    e2 = _ffn_experts(r2, Wg, Wu, Wd, 2, 512)
    out, aux = _ffn0(x_flat.astype(jnp.bfloat16), wt, Wg, Wu, Wd, e1, e2, part,
                     512, 1024)
    return out.reshape(B, T, D), aux.reshape(())


# speedup vs baseline: 1.0052x; 1.0033x over previous
import functools

import jax
import jax.numpy as jnp
from jax import lax
from jax.experimental import pallas as pl
from jax.experimental.pallas import tpu as pltpu
from jax.experimental.pallas import tpu_sc as plsc

D_MODEL = 1024
N_SCALES = 3
N_EXP = 4
N_TOTAL = 12
TOP_K = 2
D_FFN = 2752


def _logits_kernel(x_ref, wr_ref, lt_ref):
    logits = jnp.dot(x_ref[...], wr_ref[...],
                     preferred_element_type=jnp.float32)
    lt_ref[...] = logits.T


def _logits(x_flat, wr):
    n_tok = x_flat.shape[0]
    blk = 2048
    return pl.pallas_call(
        _logits_kernel,
        grid=(n_tok // blk,),
        in_specs=[
            pl.BlockSpec((blk, D_MODEL), lambda i: (i, 0)),
            pl.BlockSpec((D_MODEL, N_TOTAL), lambda i: (0, 0)),
        ],
        out_specs=pl.BlockSpec((N_TOTAL, blk), lambda i: (0, i)),
        out_shape=jax.ShapeDtypeStruct((N_TOTAL, n_tok), jnp.float32),
    )(x_flat, wr)


_SC_NW = 32
_SC_TPW = 128
_SC_G = _SC_TPW // 16


def _sc_router_body(lt_hbm, w_hbm, part_hbm, lbuf, wbuf, pbuf):
    c = lax.axis_index("c")
    s = lax.axis_index("s")
    wid = s * 2 + c
    t0 = pl.multiple_of(wid * _SC_TPW, _SC_TPW)
    pltpu.sync_copy(lt_hbm.at[:, pl.ds(t0, _SC_TPW)], lbuf)

    iota = lax.iota(jnp.int32, 16)
    acc_load = [jnp.zeros((16,), jnp.float32) for _ in range(N_TOTAL)]
    acc_cnt = [jnp.zeros((16,), jnp.float32) for _ in range(N_TOTAL)]
    for g in range(_SC_G):
        ls = [lbuf[e, pl.ds(g * 16, 16)] for e in range(N_TOTAL)]
        m = ls[0]
        for e in range(1, N_TOTAL):
            m = jnp.maximum(m, ls[e])
        es = [jnp.exp(ls[e] - m) for e in range(N_TOTAL)]
        z = es[0]
        for e in range(1, N_TOTAL):
            z = z + es[e]
        zi = 1.0 / z
        e1 = es[0]
        i1 = jnp.zeros((16,), jnp.int32)
        e2 = jnp.zeros((16,), jnp.float32)
        i2 = jnp.full((16,), N_TOTAL, jnp.int32)
        for e in range(1, N_TOTAL):
            v = es[e]
            gt1 = v > e1
            gt2 = v > e2
            i2 = jnp.where(gt1, i1, jnp.where(gt2, e, i2))
            e2 = jnp.where(gt1, e1, jnp.where(gt2, v, e2))
            i1 = jnp.where(gt1, e, i1)
            e1 = jnp.where(gt1, v, e1)
        d = e1 + e2 + 1e-8 * z
        w1 = e1 / d
        w2 = e2 / d
        for e in range(N_TOTAL):
            we = (jnp.where(i1 == e, w1, 0.0) + jnp.where(i2 == e, w2, 0.0))
            wbuf[e, pl.ds(g * 16, 16)] = we
            acc_load[e] = acc_load[e] + es[e] * zi
            acc_cnt[e] = (acc_cnt[e]
                          + jnp.where(i1 == e, 1.0, 0.0)
                          + jnp.where(i2 == e, 1.0, 0.0))

    for e in range(N_TOTAL):
        pbuf[0, pl.ds(e * 16, 16)] = acc_load[e]
        pbuf[1, pl.ds(e * 16, 16)] = acc_cnt[e]
    pltpu.sync_copy(wbuf, w_hbm.at[:, pl.ds(t0, _SC_TPW)])
    pltpu.sync_copy(pbuf, part_hbm.at[wid])


def _sc_router(ltr):
    n_tok = ltr.shape[1]
    return pl.kernel(
        _sc_router_body,
        out_type=[
            jax.ShapeDtypeStruct((N_TOTAL, n_tok), jnp.float32),
            jax.ShapeDtypeStruct((_SC_NW, 2, N_TOTAL * 16), jnp.float32),
        ],
        mesh=plsc.VectorSubcoreMesh(core_axis_name="c", subcore_axis_name="s",
                                    num_cores=2, num_subcores=16),
        scratch_types=[
            pltpu.VMEM((N_TOTAL, _SC_TPW), jnp.float32),
            pltpu.VMEM((N_TOTAL, _SC_TPW), jnp.float32),
            pltpu.VMEM((2, N_TOTAL * 16), jnp.float32),
        ],
    )(ltr)


def _pools_kernel(x1_ref, wp1_ref, x2_ref, wp2_ref, o1_ref, o2_ref,
                  acc1_ref, acc2_ref):
    k = pl.program_id(0)

    @pl.when(k == 0)
    def _():
        acc1_ref[...] = jnp.zeros_like(acc1_ref)
        acc2_ref[...] = jnp.zeros_like(acc2_ref)

    @pl.when(k < 4)
    def _():
        xb = x1_ref[...].astype(jnp.bfloat16)
        wb = wp1_ref[...].astype(jnp.bfloat16)
        acc1_ref[...] += jnp.dot(xb, wb, preferred_element_type=jnp.float32)

    @pl.when(k == 3)
    def _():
        o1_ref[...] = acc1_ref[...].astype(jnp.bfloat16)

    @pl.when(k >= 4)
    def _():
        xb = x2_ref[...].astype(jnp.bfloat16)
        wb = wp2_ref[...].astype(jnp.bfloat16)
        acc2_ref[...] += jnp.dot(xb, wb, preferred_element_type=jnp.float32)

    @pl.when(k == 19)
    def _():
        o2_ref[...] = acc2_ref[...].astype(jnp.bfloat16)


def _pools(x1, wp1, x2, wp2):
    m1 = x1.shape[0]
    m2 = x2.shape[0]
    return pl.pallas_call(
        _pools_kernel,
        grid=(20,),
        in_specs=[
            pl.BlockSpec((m1, 1024), lambda k: (0, jnp.minimum(k, 3))),
            pl.BlockSpec((1024, D_MODEL), lambda k: (jnp.minimum(k, 3), 0)),
            pl.BlockSpec((m2, 1024), lambda k: (0, jnp.maximum(k - 4, 0))),
            pl.BlockSpec((1024, D_MODEL), lambda k: (jnp.maximum(k - 4, 0), 0)),
        ],
        out_specs=[
            pl.BlockSpec((m1, D_MODEL), lambda k: (0, 0)),
            pl.BlockSpec((m2, D_MODEL), lambda k: (0, 0)),
        ],
        out_shape=[
            jax.ShapeDtypeStruct((m1, D_MODEL), jnp.bfloat16),
            jax.ShapeDtypeStruct((m2, D_MODEL), jnp.bfloat16),
        ],
        scratch_shapes=[pltpu.VMEM((m1, D_MODEL), jnp.float32),
                        pltpu.VMEM((m2, D_MODEL), jnp.float32)],
    )(x1, wp1, x2, wp2)


def _masked_weights(nb, n, wg_ref, wu_ref, wd_ref):
    if D_FFN % nb == 0:
        return (wg_ref[0, 0].astype(jnp.bfloat16),
                wu_ref[0, 0].astype(jnp.bfloat16),
                wd_ref[0, 0].astype(jnp.bfloat16))
    lim = D_FFN - n * nb
    col = jax.lax.broadcasted_iota(jnp.int32, (D_MODEL, nb), 1)
    row = jax.lax.broadcasted_iota(jnp.int32, (nb, D_MODEL), 0)
    return (jnp.where(col < lim, wg_ref[0, 0], 0.0).astype(jnp.bfloat16),
            jnp.where(col < lim, wu_ref[0, 0], 0.0).astype(jnp.bfloat16),
            jnp.where(row < lim, wd_ref[0, 0], 0.0).astype(jnp.bfloat16))


def _ffn_kernel(n_blocks, nb, r_ref, wg_ref, wu_ref, wd_ref, o_ref, acc_ref):
    n = pl.program_id(1)
    wg, wu, wd = _masked_weights(nb, n, wg_ref, wu_ref, wd_ref)

    r = r_ref[...]
    a = jnp.dot(r, wg, preferred_element_type=jnp.float32)
    b = jnp.dot(r, wu, preferred_element_type=jnp.float32)
    h = (a * jax.nn.sigmoid(a) * b).astype(jnp.bfloat16)
    part = jnp.dot(h, wd, preferred_element_type=jnp.float32)

    @pl.when(n == 0)
    def _():
        acc_ref[...] = jnp.zeros_like(acc_ref)

    acc_ref[...] += part

    @pl.when(n == n_blocks - 1)
    def _():
        o_ref[0] = acc_ref[...].astype(jnp.bfloat16)


def _ffn_experts(r, wg, wu, wd, scale_idx, nb):
    m = r.shape[0]
    n_blocks = (D_FFN + nb - 1) // nb
    return pl.pallas_call(
        functools.partial(_ffn_kernel, n_blocks, nb),
        grid=(N_EXP, n_blocks),
        in_specs=[
            pl.BlockSpec((m, D_MODEL), lambda e, n: (0, 0)),
            pl.BlockSpec((1, 1, D_MODEL, nb), lambda e, n: (scale_idx, e, 0, n)),
            pl.BlockSpec((1, 1, D_MODEL, nb), lambda e, n: (scale_idx, e, 0, n)),
            pl.BlockSpec((1, 1, nb, D_MODEL), lambda e, n: (scale_idx, e, n, 0)),
        ],
        out_specs=pl.BlockSpec((1, m, D_MODEL), lambda e, n: (e, 0, 0)),
        out_shape=jax.ShapeDtypeStruct((N_EXP, m, D_MODEL), jnp.bfloat16),
        scratch_shapes=[pltpu.VMEM((m, D_MODEL), jnp.float32)],
    )(r, wg, wu, wd)


def _ffn0_kernel(n_blocks, nb, mb, n_tok, x_ref, w_ref, wg_ref, wu_ref, wd_ref,
                 e1_ref, e2_ref, part_ref, o_ref, aux_ref, wtok_ref):
    i = pl.program_id(0)
    e = pl.program_id(1)
    n = pl.program_id(2)

    @pl.when(jnp.logical_and(i == 0, jnp.logical_and(e == 0, n == 0)))
    def _():
        p = jnp.sum(part_ref[...], axis=0)
        r192 = jax.lax.broadcasted_iota(jnp.int32, (N_TOTAL * 16, N_TOTAL), 0)
        c192 = jax.lax.broadcasted_iota(jnp.int32, (N_TOTAL * 16, N_TOTAL), 1)
        sel = (r192 // 16 == c192).astype(jnp.float32)
        sums = jnp.dot(p, sel, preferred_element_type=jnp.float32)
        load = sums[0:1, :] / n_tok
        frac = sums[1:2, :] / (n_tok * TOP_K)
        aux_ref[...] = (N_TOTAL * jnp.sum(frac * load)).reshape(1, 1)

    @pl.when(jnp.logical_and(e == 0, n == 0))
    def _():
        wtok_ref[...] = w_ref[...].T

    wg, wu, wd = _masked_weights(nb, n, wg_ref, wu_ref, wd_ref)

    x = x_ref[...]
    wtok = wtok_ref[...]
    lane = jax.lax.broadcasted_iota(jnp.int32, wtok.shape, 1)
    wcol = jnp.sum(jnp.where(lane == e, wtok, 0.0), axis=1, keepdims=True)
    a = jnp.dot(x, wg, preferred_element_type=jnp.float32)
    b = jnp.dot(x, wu, preferred_element_type=jnp.float32)
    h = (a * jax.nn.sigmoid(a) * b * wcol).astype(jnp.bfloat16)
    part = jnp.dot(h, wd, preferred_element_type=jnp.float32)

    @pl.when(jnp.logical_and(e == 0, n == 0))
    def _():
        g1, g2 = mb // 4, mb // 16
        wtok = wtok_ref[...]
        acc = jnp.zeros((mb, D_MODEL), jnp.float32)
        for ee in range(N_EXP):
            v1 = e1_ref[ee].astype(jnp.float32)
            v1r = jnp.broadcast_to(v1[:, None, :], (g1, 4, D_MODEL))
            v1r = v1r.reshape(mb, D_MODEL)
            acc += wtok[:, N_EXP + ee:N_EXP + ee + 1] * v1r
            v2 = e2_ref[ee].astype(jnp.float32)
            v2r = jnp.broadcast_to(v2[:, None, :], (g2, 16, D_MODEL))
            v2r = v2r.reshape(mb, D_MODEL)
            acc += wtok[:, 2 * N_EXP + ee:2 * N_EXP + ee + 1] * v2r
        o_ref[...] = acc.astype(jnp.bfloat16)

    o_ref[...] += part.astype(jnp.bfloat16)


def _ffn0(xb, w, wg, wu, wd, e1, e2, part, nb, mb):
    m = xb.shape[0]
    n_blocks = (D_FFN + nb - 1) // nb
    return pl.pallas_call(
        functools.partial(_ffn0_kernel, n_blocks, nb, mb, m),
        grid=(m // mb, N_EXP, n_blocks),
        in_specs=[
            pl.BlockSpec((mb, D_MODEL), lambda i, e, n: (i, 0)),
            pl.BlockSpec((N_TOTAL, mb), lambda i, e, n: (0, i)),
            pl.BlockSpec((1, 1, D_MODEL, nb), lambda i, e, n: (0, e, 0, n)),
            pl.BlockSpec((1, 1, D_MODEL, nb), lambda i, e, n: (0, e, 0, n)),
            pl.BlockSpec((1, 1, nb, D_MODEL), lambda i, e, n: (0, e, n, 0)),
            pl.BlockSpec((N_EXP, mb // 4, D_MODEL), lambda i, e, n: (0, i, 0)),
            pl.BlockSpec((N_EXP, mb // 16, D_MODEL), lambda i, e, n: (0, i, 0)),
            pl.BlockSpec((_SC_NW, 2, N_TOTAL * 16), lambda i, e, n: (0, 0, 0)),
        ],
        out_specs=[
            pl.BlockSpec((mb, D_MODEL), lambda i, e, n: (i, 0)),
            pl.BlockSpec((1, 1), lambda i, e, n: (0, 0)),
        ],
        out_shape=[
            jax.ShapeDtypeStruct((m, D_MODEL), jnp.bfloat16),
            jax.ShapeDtypeStruct((1, 1), jnp.float32),
        ],
        scratch_shapes=[pltpu.VMEM((mb, N_TOTAL), jnp.float32)],
    )(xb, w, wg, wu, wd, e1, e2, part)


def kernel(x, Wr, Wp1, Wp2, Wg, Wu, Wd):
    B, T, D = x.shape
    n_tok = B * T
    x_flat = x.reshape(n_tok, D)
    x1 = x.reshape(n_tok // 4, 4 * D)
    x2 = x.reshape(n_tok // 16, 16 * D)

    ltr = _logits(x_flat, Wr)
    wt, part = _sc_router(ltr)
    r1, r2 = _pools(x1, Wp1, x2, Wp2)
    e1 = _ffn_experts(r1, Wg, Wu, Wd, 1, 512)
    e2 = _ffn_experts(r2, Wg, Wu, Wd, 2, 512)
    out, aux = _ffn0(x_flat.astype(jnp.bfloat16), wt, Wg, Wu, Wd, e1, e2, part,
                     512, 1024)
    return out.astype(jnp.float32).reshape(B, T, D), aux.reshape(())
